# baseline (speedup 1.0000x reference)
import functools

import jax
import jax.numpy as jnp
from jax import lax
from jax.experimental import pallas as pl
from jax.experimental.pallas import tpu as pltpu
from jax.experimental.pallas import tpu_sc as plsc

B = 16384
CTX = 20
D = 64
L = 16
NC = 2
NS = 16
NW = NC * NS
ROWS_PER_W = B // NW
CHUNK = 32
N_CHUNKS = ROWS_PER_W // CHUNK
IDX_PER_CHUNK = CHUNK * CTX
G = 128
NG = IDX_PER_CHUNK // G
VP = 500000


def _make_kernel():
    mesh = plsc.VectorSubcoreMesh(
        core_axis_name="c", subcore_axis_name="s", num_cores=NC, num_subcores=NS
    )

    @functools.partial(
        pl.kernel,
        out_type=jax.ShapeDtypeStruct((B, D), jnp.float32),
        mesh=mesh,
        compiler_params=pltpu.CompilerParams(needs_layout_passes=False),
        scratch_types=[
            pltpu.VMEM((IDX_PER_CHUNK,), jnp.int32),
            pltpu.VMEM((IDX_PER_CHUNK,), jnp.int32),
            pltpu.VMEM((IDX_PER_CHUNK, 2 * D), jnp.float32),
            pltpu.VMEM((CHUNK, D), jnp.float32),
            pltpu.SemaphoreType.DMA,
        ],
    )
    def cbow_kernel(g_hbm, o_hbm, table_hbm, out_hbm, idx_v, off_v,
                    rows_v, out_v, sem):
        wid = lax.axis_index("s") * NC + lax.axis_index("c")
        base = wid * ROWS_PER_W

        def chunk_body(ci, carry):
            cbase = base + ci * CHUNK
            pltpu.sync_copy(g_hbm.at[pl.ds(cbase * CTX, IDX_PER_CHUNK)], idx_v)
            pltpu.sync_copy(o_hbm.at[pl.ds(cbase * CTX, IDX_PER_CHUNK)], off_v)
            descs = [
                pltpu.async_copy(
                    table_hbm.at[idx_v.at[pl.ds(g * G, G)]],
                    rows_v.at[pl.ds(g * G, G)],
                    sem,
                )
                for g in range(NG)
            ]
            for d in descs:
                d.wait()

            lanes = lax.iota(jnp.int32, L)

            def red_body(b, carry2):
                r0 = b * CTX
                accs = [jnp.zeros((L,), jnp.float32) for _ in range(D // L)]
                for j in range(CTX):
                    row = jnp.full((L,), r0 + j, jnp.int32)
                    off = plsc.load_gather(off_v, [row])
                    col0 = off + lanes
                    for k in range(D // L):
                        accs[k] = accs[k] + plsc.load_gather(
                            rows_v, [row, col0 + (k * L)])
                for k in range(D // L):
                    out_v[b, pl.ds(k * L, L)] = accs[k] * jnp.float32(1.0 / CTX)
                return carry2

            lax.fori_loop(0, CHUNK, red_body, 0)
            pltpu.sync_copy(out_v, out_hbm.at[pl.ds(cbase, CHUNK)])
            return carry

        lax.fori_loop(0, N_CHUNKS, chunk_body, 0)

    return cbow_kernel


_cbow = _make_kernel()


@jax.jit
def kernel(x, y, table):
    del y
    x_flat = x.astype(jnp.int32).reshape(B * CTX)
    g_flat = x_flat >> 1
    o_flat = (x_flat & 1) * D
    table_pairs = table.reshape(VP, 2 * D)
    return _cbow(g_flat, o_flat, table_pairs)
